# baseline (speedup 1.0000x reference)
import functools

import jax
import jax.numpy as jnp
from jax import lax
from jax.experimental import pallas as pl
from jax.experimental.pallas import tpu as pltpu
from jax.experimental.pallas import tpu_sc as plsc

EMBED = 768
MAX_POS = 64
LANES = 16
CHUNK = 32
NBUF = 3


@functools.cache
def _make_kernel(n_rows):
    info = plsc.get_sparse_core_info()
    nc, ns = info.num_cores, info.num_subcores
    nw = nc * ns
    rows_per_w = n_rows // nw
    n_chunks = rows_per_w // CHUNK
    n_main = n_chunks - 1
    n_groups = n_main // NBUF
    assert n_groups * NBUF == n_main
    mesh = plsc.VectorSubcoreMesh(core_axis_name="c", subcore_axis_name="s")

    @functools.partial(
        pl.kernel,
        out_type=jax.ShapeDtypeStruct((n_rows, EMBED), jnp.float32),
        mesh=mesh,
        scratch_types=(
            [pltpu.VMEM((CHUNK,), jnp.int32)] * NBUF
            + [pltpu.VMEM((CHUNK, EMBED), jnp.float32)] * NBUF
            + [pltpu.VMEM((MAX_POS, EMBED), jnp.float32)]
            + [pltpu.SemaphoreType.DMA] * (3 * NBUF)
        ),
    )
    def k(ids_hbm, table_hbm, pos_hbm, out_hbm, *scratch):
        idx = scratch[:NBUF]
        rows = scratch[NBUF:2 * NBUF]
        pos_v = scratch[2 * NBUF]
        semg = scratch[2 * NBUF + 1:2 * NBUF + 1 + NBUF]
        semi = scratch[2 * NBUF + 1 + NBUF:2 * NBUF + 1 + 2 * NBUF]
        semw = scratch[2 * NBUF + 1 + 2 * NBUF:]

        wid = lax.axis_index("s") * nc + lax.axis_index("c")
        base = wid * rows_per_w
        cbase = wid * n_chunks
        pltpu.sync_copy(pos_hbm, pos_v)

        def id_load(c, b):
            return pltpu.make_async_copy(ids_hbm.at[cbase + c], idx[b], semi[b])

        def gather(b):
            return pltpu.make_async_copy(table_hbm.at[idx[b]], rows[b], semg[b])

        def writeout(c, b):
            return pltpu.make_async_copy(
                rows[b], out_hbm.at[pl.ds(base + c * CHUNK, CHUNK)], semw[b]
            )

        def addpos(b, c):
            off = lax.bitwise_and(c, 1) * CHUNK

            def row(r, carry):
                pr = off + r
                for d in range(EMBED // LANES):
                    sl = pl.ds(d * LANES, LANES)
                    plsc.addupdate(rows[b].at[r, sl], pos_v[pr, sl])
                return carry

            lax.fori_loop(0, CHUNK, row, 0)

        id_load(0, 0).start()
        id_load(0, 0).wait()
        gather(0).start()
        id_load(1, 1).start()
        id_load(1, 1).wait()
        gather(1).start()
        id_load(2, 2).start()

        def group_body(i, carry):
            for j in range(NBUF):
                c = NBUF * i + j
                gather(j).wait()

                @pl.when(c + NBUF < n_chunks)
                def _():
                    id_load(c + NBUF, j).start()

                addpos(j, c)
                writeout(c, j).start()

                b = (j + 2) % NBUF
                if j == 0:
                    @pl.when(i >= 1)
                    def _():
                        writeout(c - 1, b).wait()
                else:
                    writeout(c - 1, b).wait()

                @pl.when(c + 2 < n_chunks)
                def _():
                    id_load(c + 2, b).wait()
                    gather(b).start()

            return carry

        lax.fori_loop(0, n_groups, group_body, 0)

        last = n_chunks - 1
        gather(0).wait()
        addpos(0, last)
        writeout(last, 0).start()
        writeout(last - 1, 2).wait()
        writeout(last, 0).wait()

    return k


def kernel(input_ids, token_embedding, position_embedding):
    b, s = input_ids.shape
    n_rows = b * s
    ids2 = input_ids.reshape(n_rows // CHUNK, CHUNK).astype(jnp.int32)
    out = _make_kernel(n_rows)(ids2, token_embedding, position_embedding)
    return out.reshape(b, s, EMBED)


# speedup vs baseline: 1.4642x; 1.4642x over previous
import functools

import jax
import jax.numpy as jnp
from jax import lax
from jax.experimental import pallas as pl
from jax.experimental.pallas import tpu as pltpu
from jax.experimental.pallas import tpu_sc as plsc

EMBED = 768
MAX_POS = 64
LANES = 16
CHUNK = 16
NBUF = 4


@functools.cache
def _make_kernel(n_rows):
    info = plsc.get_sparse_core_info()
    nc, ns = info.num_cores, info.num_subcores
    nw = nc * ns
    rows_per_w = n_rows // nw
    n_chunks = rows_per_w // CHUNK
    n_groups = n_chunks // NBUF
    assert n_groups * NBUF == n_chunks
    mesh = plsc.VectorSubcoreMesh(core_axis_name="c", subcore_axis_name="s")

    @functools.partial(
        pl.kernel,
        out_type=jax.ShapeDtypeStruct((n_rows, EMBED), jnp.float32),
        mesh=mesh,
        scratch_types=(
            [pltpu.VMEM((CHUNK,), jnp.int32)] * NBUF
            + [pltpu.VMEM((CHUNK, EMBED), jnp.float32)] * NBUF
            + [pltpu.VMEM((MAX_POS, EMBED), jnp.float32)]
            + [pltpu.SemaphoreType.DMA] * (3 * NBUF)
        ),
    )
    def k(ids_hbm, table_hbm, pos_hbm, out_hbm, *scratch):
        idx = scratch[:NBUF]
        rows = scratch[NBUF:2 * NBUF]
        pos_v = scratch[2 * NBUF]
        semg = scratch[2 * NBUF + 1:3 * NBUF + 1]
        semi = scratch[3 * NBUF + 1:4 * NBUF + 1]
        semw = scratch[4 * NBUF + 1:]

        wid = lax.axis_index("s") * nc + lax.axis_index("c")
        base = wid * rows_per_w
        cbase = wid * n_chunks
        pltpu.sync_copy(pos_hbm, pos_v)

        def id_load(c, b):
            return pltpu.make_async_copy(ids_hbm.at[cbase + c], idx[b], semi[b])

        def gather(b):
            return pltpu.make_async_copy(table_hbm.at[idx[b]], rows[b], semg[b])

        def writeout(c, b):
            return pltpu.make_async_copy(
                rows[b], out_hbm.at[pl.ds(base + c * CHUNK, CHUNK)], semw[b]
            )

        def addpos(b, off):
            def row(r, carry):
                for d in range(EMBED // LANES):
                    sl = pl.ds(d * LANES, LANES)
                    plsc.addupdate(rows[b].at[r, sl], pos_v[off + r, sl])
                return carry

            lax.fori_loop(0, CHUNK, row, 0)

        id_load(0, 0).start()
        id_load(0, 0).wait()
        gather(0).start()
        id_load(1, 1).start()
        id_load(1, 1).wait()
        gather(1).start()
        id_load(2, 2).start()
        id_load(3, 3).start()

        def group_body(i, carry):
            for j in range(NBUF):
                c = NBUF * i + j
                gather(j).wait()

                @pl.when(c + NBUF < n_chunks)
                def _():
                    id_load(c + NBUF, j).start()

                addpos(j, j * CHUNK)
                writeout(c, j).start()

                b = (j + 2) % NBUF
                if j < 2:
                    @pl.when(i >= 1)
                    def _():
                        writeout(c - 2, b).wait()
                else:
                    writeout(c - 2, b).wait()

                @pl.when(c + 2 < n_chunks)
                def _():
                    id_load(c + 2, b).wait()
                    gather(b).start()

            return carry

        lax.fori_loop(0, n_groups, group_body, 0)

        writeout(n_chunks - 2, 2).wait()
        writeout(n_chunks - 1, 3).wait()

    return k


def kernel(input_ids, token_embedding, position_embedding):
    b, s = input_ids.shape
    n_rows = b * s
    ids2 = input_ids.reshape(n_rows // CHUNK, CHUNK).astype(jnp.int32)
    out = _make_kernel(n_rows)(ids2, token_embedding, position_embedding)
    return out.reshape(b, s, EMBED)
